# baseline (speedup 1.0000x reference)
import jax
import jax.numpy as jnp
from jax import lax
from jax.experimental import pallas as pl
from jax.experimental.pallas import tpu as pltpu

N_TOK = 32768
D_MODEL = 768
N_EXP = 64
_BT = 4096
_KS = 2
_DK = D_MODEL // _KS


def _gate_body(x_ref, w_ref, idx_ref, gate_ref, acc_ref):
    k = pl.program_id(1)
    partial = lax.dot_general(
        w_ref[...], x_ref[...],
        (((1,), (1,)), ((), ())),
        preferred_element_type=jnp.float32,
    )

    @pl.when(k == 0)
    def _():
        acc_ref[...] = partial

    @pl.when(k == _KS - 1)
    def _():
        logits = acc_ref[...] + partial
        m = jnp.max(logits, axis=0, keepdims=True)
        ii = lax.broadcasted_iota(jnp.int32, (N_EXP, _BT), 0)
        cand = jnp.where(logits == m, ii, N_EXP)
        idx_ref[...] = jnp.min(cand, axis=0, keepdims=True)
        s = jnp.sum(jnp.exp(logits - m), axis=0, keepdims=True)
        gate_ref[...] = 1.0 / s


def kernel(x, W):
    idx2, gate2 = pl.pallas_call(
        _gate_body,
        grid=(N_TOK // _BT, _KS),
        in_specs=[
            pl.BlockSpec((_BT, _DK), lambda i, k: (i, k)),
            pl.BlockSpec((N_EXP, _DK), lambda i, k: (0, k)),
        ],
        out_specs=[
            pl.BlockSpec((1, _BT), lambda i, k: (0, i)),
            pl.BlockSpec((1, _BT), lambda i, k: (0, i)),
        ],
        out_shape=[
            jax.ShapeDtypeStruct((1, N_TOK), jnp.int32),
            jax.ShapeDtypeStruct((1, N_TOK), jnp.float32),
        ],
        scratch_shapes=[
            pltpu.VMEM((N_EXP, _BT), jnp.float32),
        ],
    )(x, W)
    expert_indices = idx2.reshape(N_TOK)
    expert_gates = gate2.reshape(N_TOK)
    load_balance_loss = jnp.zeros((), jnp.float32)
    return (expert_indices, expert_gates, load_balance_loss)


# speedup vs baseline: 1.0851x; 1.0851x over previous
import jax
import jax.numpy as jnp
from jax import lax
from jax.experimental import pallas as pl
from jax.experimental.pallas import tpu as pltpu

N_TOK = 32768
D_MODEL = 768
N_EXP = 64

_SIZES = (512, 512, 1024, 2048, 4096, 4096, 4096, 4096, 4096, 4096, 4096)
_OFFS = tuple(sum(_SIZES[:i]) for i in range(len(_SIZES)))
assert sum(_SIZES) == N_TOK
_NBUF = 3
_BUFROWS = 4096


def _gate_body(x_hbm, w_ref, idx_ref, gate_ref, bufs, sems):
    w = w_ref[...]

    def start(c):
        b = c % _NBUF
        pltpu.make_async_copy(
            x_hbm.at[pl.ds(_OFFS[c], _SIZES[c]), :],
            bufs.at[b, pl.ds(0, _SIZES[c]), :],
            sems.at[b],
        ).start()

    def compute(c):
        b = c % _NBUF
        n = _SIZES[c]
        pltpu.make_async_copy(
            x_hbm.at[pl.ds(_OFFS[c], n), :],
            bufs.at[b, pl.ds(0, n), :],
            sems.at[b],
        ).wait()
        logits = lax.dot_general(
            w, bufs[b, pl.ds(0, n), :],
            (((1,), (1,)), ((), ())),
            preferred_element_type=jnp.float32,
        )
        m = jnp.max(logits, axis=0, keepdims=True)
        ii = lax.broadcasted_iota(jnp.int32, (N_EXP, n), 0)
        cand = jnp.where(logits == m, ii, N_EXP)
        idx_ref[:, pl.ds(_OFFS[c], n)] = jnp.min(cand, axis=0, keepdims=True)
        s = jnp.sum(jnp.exp(logits - m), axis=0, keepdims=True)
        gate_ref[:, pl.ds(_OFFS[c], n)] = 1.0 / s

    nc = len(_SIZES)
    lead = _NBUF - 1
    for c in range(lead):
        start(c)
    for c in range(nc):
        if c + lead < nc:
            start(c + lead)
        compute(c)


def kernel(x, W):
    idx2, gate2 = pl.pallas_call(
        _gate_body,
        in_specs=[
            pl.BlockSpec(memory_space=pl.ANY),
            pl.BlockSpec((N_EXP, D_MODEL), lambda: (0, 0)),
        ],
        out_specs=[
            pl.BlockSpec((1, N_TOK), lambda: (0, 0)),
            pl.BlockSpec((1, N_TOK), lambda: (0, 0)),
        ],
        out_shape=[
            jax.ShapeDtypeStruct((1, N_TOK), jnp.int32),
            jax.ShapeDtypeStruct((1, N_TOK), jnp.float32),
        ],
        scratch_shapes=[
            pltpu.VMEM((_NBUF, _BUFROWS, D_MODEL), jnp.float32),
            pltpu.SemaphoreType.DMA((_NBUF,)),
        ],
    )(x, W)
    expert_indices = idx2.reshape(N_TOK)
    expert_gates = gate2.reshape(N_TOK)
    load_balance_loss = jnp.zeros((), jnp.float32)
    return (expert_indices, expert_gates, load_balance_loss)


# speedup vs baseline: 1.1829x; 1.0901x over previous
import jax
import jax.numpy as jnp
from jax import lax
from jax.experimental import pallas as pl

N_TOK = 32768
D_MODEL = 768
N_EXP = 64
_BT = 4096


def _gate_body(x_ref, w_ref, idx_ref, gate_ref):
    logits = lax.dot_general(
        w_ref[...], x_ref[...],
        (((1,), (1,)), ((), ())),
        preferred_element_type=jnp.float32,
    )
    m = jnp.max(logits, axis=0, keepdims=True)
    ii = lax.broadcasted_iota(jnp.int32, (N_EXP, _BT), 0)
    cand = jnp.where(logits == m, ii, N_EXP)
    idx = jnp.min(cand, axis=0, keepdims=True)
    s = jnp.sum(jnp.exp(logits - m), axis=0, keepdims=True)
    idx_ref[...] = idx
    gate_ref[...] = 1.0 / s


def kernel(x, W):
    idx2, gate2 = pl.pallas_call(
        _gate_body,
        grid=(N_TOK // _BT,),
        in_specs=[
            pl.BlockSpec((_BT, D_MODEL), lambda i: (i, 0)),
            pl.BlockSpec((N_EXP, D_MODEL), lambda i: (0, 0)),
        ],
        out_specs=[
            pl.BlockSpec((1, _BT), lambda i: (0, i)),
            pl.BlockSpec((1, _BT), lambda i: (0, i)),
        ],
        out_shape=[
            jax.ShapeDtypeStruct((1, N_TOK), jnp.int32),
            jax.ShapeDtypeStruct((1, N_TOK), jnp.float32),
        ],
    )(x, W)
    expert_indices = idx2.reshape(N_TOK)
    expert_gates = gate2.reshape(N_TOK)
    load_balance_loss = jnp.zeros((), jnp.float32)
    return (expert_indices, expert_gates, load_balance_loss)
